# baseline (speedup 1.0000x reference)
import functools

import jax
import jax.numpy as jnp
from jax import lax
from jax.experimental import pallas as pl
from jax.experimental.pallas import tpu as pltpu
from jax.experimental.pallas import tpu_sc as plsc

N = 10000
M = 128
MY = 16
E = 320000
MAX_ITER = 30
GAMMA = 0.8
EPS_F = 1e-6

N_PAD = 10240
CH = 128
N_TILES = 32
CPT = 80
E_PAD = N_TILES * CPT * CH
E_ALLOC = E_PAD + 2 * CH
RPT = N_PAD // 16

_MESH = plsc.VectorSubcoreMesh(core_axis_name="c", subcore_axis_name="s")


@functools.partial(
    pl.kernel,
    out_type=jax.ShapeDtypeStruct((4 * N_PAD, M), jnp.float32),
    mesh=_MESH,
    scratch_types=[
        pltpu.VMEM_SHARED((N_PAD, M), jnp.float32),
        pltpu.VMEM((CH,), jnp.int32),
        pltpu.VMEM((CH, M), jnp.float32),
        pltpu.VMEM((CH, M), jnp.float32),
    ],
)
def _deg_kernel(srcp, dstp, deg_out, acc, idx_v, ones_v, bnc_v):
    cid = lax.axis_index("c")
    sid = lax.axis_index("s")
    wid = sid * 2 + cid
    onerow = jnp.ones((16,), jnp.float32)
    zrow = jnp.zeros((16,), jnp.float32)

    def fill_ones(r, carry):
        for k in range(M // 16):
            ones_v[r, pl.ds(k * 16, 16)] = onerow
        return carry

    lax.fori_loop(0, CH, fill_ones, 0)

    def fill_bnc_zero(r, carry):
        for k in range(M // 16):
            bnc_v[r, pl.ds(k * 16, 16)] = zrow
        return carry

    base_r = sid * RPT

    def zero_acc(b, carry):
        pltpu.sync_copy(bnc_v, acc.at[pl.ds(base_r + b * CH, CH)])
        return carry

    def count_phase(idx_hbm, out_base):
        lax.fori_loop(0, CH, fill_bnc_zero, 0)
        lax.fori_loop(0, RPT // CH, zero_acc, 0)
        plsc.subcore_barrier()

        def edge_step(j, carry):
            base = (wid * CPT + j) * CH
            pltpu.sync_copy(idx_hbm.at[pl.ds(base, CH)], idx_v)
            pltpu.sync_copy(ones_v, acc.at[idx_v], add=True)
            return carry

        lax.fori_loop(0, CPT, edge_step, 0)
        plsc.subcore_barrier()

        def writeback(b, carry):
            off = base_r + b * CH
            pltpu.sync_copy(acc.at[pl.ds(off, CH)], bnc_v)
            pltpu.sync_copy(bnc_v, deg_out.at[pl.ds(out_base + off, CH)])
            return carry

        lax.fori_loop(0, RPT // CH, writeback, 0)
        plsc.subcore_barrier()

    count_phase(srcp, cid * N_PAD)
    count_phase(dstp, 2 * N_PAD + cid * N_PAD)


CH2 = 256
CPT2 = CPT * CH // CH2


@functools.partial(
    pl.kernel,
    out_type=jax.ShapeDtypeStruct((2 * N_PAD, M), jnp.float32),
    mesh=_MESH,
    scratch_types=[
        pltpu.VMEM_SHARED((N_PAD, M), jnp.float32),
        pltpu.VMEM((CH2,), jnp.int32),
        pltpu.VMEM((CH2,), jnp.int32),
        pltpu.VMEM((CH2, M), jnp.float32),
        pltpu.SemaphoreType.DMA,
    ],
)
def _spmm_kernel(zt, srcp, dstp, p_out, acc, idx_s, idx_d, rows_v, sem):
    cid = lax.axis_index("c")
    sid = lax.axis_index("s")
    wid = sid * 2 + cid
    zrow = jnp.zeros((16,), jnp.float32)

    def fill_zero(r, carry):
        for k in range(M // 16):
            rows_v[r, pl.ds(k * 16, 16)] = zrow
        return carry

    lax.fori_loop(0, CH, fill_zero, 0)

    base_r = sid * RPT

    def zero_acc(b, carry):
        pltpu.sync_copy(rows_v.at[pl.ds(0, CH)],
                        acc.at[pl.ds(base_r + b * CH, CH)])
        return carry

    lax.fori_loop(0, RPT // CH, zero_acc, 0)
    plsc.subcore_barrier()

    def edge_step(j, carry):
        base = (wid * CPT2 + j) * CH2
        pltpu.sync_copy(srcp.at[pl.ds(base, CH2)], idx_s)
        pltpu.sync_copy(dstp.at[pl.ds(base, CH2)], idx_d)
        pltpu.async_copy(zt.at[idx_s], rows_v, sem).wait()
        pltpu.sync_copy(rows_v, acc.at[idx_d], add=True)
        return carry

    lax.fori_loop(0, CPT2, edge_step, 0)
    plsc.subcore_barrier()

    def writeback(b, carry):
        off = base_r + b * CH
        pltpu.sync_copy(acc.at[pl.ds(off, CH)], rows_v.at[pl.ds(0, CH)])
        pltpu.sync_copy(rows_v.at[pl.ds(0, CH)],
                        p_out.at[pl.ds(cid * N_PAD + off, CH)])
        return carry

    lax.fori_loop(0, RPT // CH, writeback, 0)


def _prep_body(xb, fw, do0, do1, di0, di1, xa_o, cb_o, gf_o):
    dego = do0[:, :1] + do1[:, :1]
    degi = di0[:, :1] + di1[:, :1]
    a = lax.rsqrt(jnp.maximum(dego, 1.0))
    b = lax.rsqrt(jnp.maximum(degi, 1.0))
    c = GAMMA * a * b
    xa_o[...] = xb[...].T * a
    cb_o[...] = jnp.broadcast_to(c, (M, M))
    g = lax.dot_general(fw[...], fw[...], (((0,), (0,)), ((), ())),
                        precision=lax.Precision.HIGHEST)
    nrm = jnp.sqrt(jnp.sum(g * g))
    gf_o[...] = g / (nrm + EPS_F)


def _prep_call(xp, f_w, degs):
    nb = N_PAD // M
    return pl.pallas_call(
        _prep_body,
        grid=(nb,),
        in_specs=[
            pl.BlockSpec((M, M), lambda i: (0, i)),
            pl.BlockSpec((M, M), lambda i: (0, 0)),
            pl.BlockSpec((M, M), lambda i: (i, 0)),
            pl.BlockSpec((M, M), lambda i, _nb=nb: (i + _nb, 0)),
            pl.BlockSpec((M, M), lambda i, _nb=nb: (i + 2 * _nb, 0)),
            pl.BlockSpec((M, M), lambda i, _nb=nb: (i + 3 * _nb, 0)),
        ],
        out_specs=[
            pl.BlockSpec((M, M), lambda i: (i, 0)),
            pl.BlockSpec((M, M), lambda i: (i, 0)),
            pl.BlockSpec((M, M), lambda i: (0, 0)),
        ],
        out_shape=[
            jax.ShapeDtypeStruct((N_PAD, M), jnp.float32),
            jax.ShapeDtypeStruct((N_PAD, M), jnp.float32),
            jax.ShapeDtypeStruct((M, M), jnp.float32),
        ],
        compiler_params=pltpu.CompilerParams(
            dimension_semantics=("arbitrary",)),
    )(xp, f_w, degs, degs, degs, degs)


def _update_body(p0, p1, cbk, xak, gf, z_o):
    acc = (p0[...] + p1[...]) * cbk[...]
    z_o[...] = lax.dot_general(
        acc, gf[...], (((1,), (0,)), ((), ())),
        precision=lax.Precision.HIGHEST) + xak[...]


def _update_call(pflat, cb, xa, gf):
    rb = 1024
    nb = N_PAD // rb
    return pl.pallas_call(
        _update_body,
        grid=(nb,),
        in_specs=[
            pl.BlockSpec((rb, M), lambda i: (i, 0)),
            pl.BlockSpec((rb, M), lambda i, _nb=nb: (i + _nb, 0)),
            pl.BlockSpec((rb, M), lambda i: (i, 0)),
            pl.BlockSpec((rb, M), lambda i: (i, 0)),
            pl.BlockSpec((M, M), lambda i: (0, 0)),
        ],
        out_specs=pl.BlockSpec((rb, M), lambda i: (i, 0)),
        out_shape=jax.ShapeDtypeStruct((N_PAD, M), jnp.float32),
        compiler_params=pltpu.CompilerParams(
            dimension_semantics=("arbitrary",)),
    )(pflat, pflat, cb, xa, gf)


def _final_body(zb, bw, o_ref):
    z = zb[...]
    nrm = jnp.maximum(jnp.sqrt(jnp.sum(z * z, axis=1, keepdims=True)), 1e-12)
    zn = z / nrm
    o_ref[...] = lax.dot_general(zn, bw[...], (((1,), (1,)), ((), ())),
                                 precision=lax.Precision.HIGHEST)


def _final_call(z, b_w):
    rb = 1000
    nb = N // rb
    return pl.pallas_call(
        _final_body,
        grid=(nb,),
        in_specs=[
            pl.BlockSpec((rb, M), lambda i: (i, 0)),
            pl.BlockSpec((MY, M), lambda i: (0, 0)),
        ],
        out_specs=pl.BlockSpec((rb, MY), lambda i: (i, 0)),
        out_shape=jax.ShapeDtypeStruct((N, MY), jnp.float32),
        compiler_params=pltpu.CompilerParams(
            dimension_semantics=("arbitrary",)),
    )(z, b_w)


def kernel(X, edge_index, F_w, B_w):
    src = edge_index[0]
    dst = edge_index[1]
    pad = jnp.full((E_ALLOC - E,), N, dtype=jnp.int32)
    srcp = jnp.concatenate([src, pad])
    dstp = jnp.concatenate([dst, pad])
    xp = jnp.pad(X, ((0, 0), (0, N_PAD - N)))

    degs = _deg_kernel(srcp, dstp)
    xa, cb, gf = _prep_call(xp, F_w, degs)

    z = xa
    for _ in range(MAX_ITER):
        pflat = _spmm_kernel(z, srcp, dstp)
        z = _update_call(pflat, cb, xa, gf)

    return _final_call(z, B_w)


# speedup vs baseline: 1.3440x; 1.3440x over previous
import functools

import jax
import jax.numpy as jnp
from jax import lax
from jax.experimental import pallas as pl
from jax.experimental.pallas import tpu as pltpu
from jax.experimental.pallas import tpu_sc as plsc

N = 10000
M = 128
MY = 16
E = 320000
MAX_ITER = 30
GAMMA = 0.8
EPS_F = 1e-6

N_PAD = 10240
CH = 128
N_TILES = 32
CPT = 80
E_PAD = N_TILES * CPT * CH
E_ALLOC = E_PAD + 2 * CH
RPT = N_PAD // 16

_MESH = plsc.VectorSubcoreMesh(core_axis_name="c", subcore_axis_name="s")


@functools.partial(
    pl.kernel,
    out_type=jax.ShapeDtypeStruct((4 * N_PAD, M), jnp.float32),
    mesh=_MESH,
    scratch_types=[
        pltpu.VMEM_SHARED((N_PAD, M), jnp.float32),
        pltpu.VMEM((CH,), jnp.int32),
        pltpu.VMEM((CH, M), jnp.float32),
        pltpu.VMEM((CH, M), jnp.float32),
    ],
)
def _deg_kernel(srcp, dstp, deg_out, acc, idx_v, ones_v, bnc_v):
    cid = lax.axis_index("c")
    sid = lax.axis_index("s")
    wid = sid * 2 + cid
    onerow = jnp.ones((16,), jnp.float32)
    zrow = jnp.zeros((16,), jnp.float32)

    def fill_ones(r, carry):
        for k in range(M // 16):
            ones_v[r, pl.ds(k * 16, 16)] = onerow
        return carry

    lax.fori_loop(0, CH, fill_ones, 0)

    def fill_bnc_zero(r, carry):
        for k in range(M // 16):
            bnc_v[r, pl.ds(k * 16, 16)] = zrow
        return carry

    base_r = sid * RPT

    def zero_acc(b, carry):
        pltpu.sync_copy(bnc_v, acc.at[pl.ds(base_r + b * CH, CH)])
        return carry

    def count_phase(idx_hbm, out_base):
        lax.fori_loop(0, CH, fill_bnc_zero, 0)
        lax.fori_loop(0, RPT // CH, zero_acc, 0)
        plsc.subcore_barrier()

        def edge_step(j, carry):
            base = (wid * CPT + j) * CH
            pltpu.sync_copy(idx_hbm.at[pl.ds(base, CH)], idx_v)
            pltpu.sync_copy(ones_v, acc.at[idx_v], add=True)
            return carry

        lax.fori_loop(0, CPT, edge_step, 0)
        plsc.subcore_barrier()

        def writeback(b, carry):
            off = base_r + b * CH
            pltpu.sync_copy(acc.at[pl.ds(off, CH)], bnc_v)
            pltpu.sync_copy(bnc_v, deg_out.at[pl.ds(out_base + off, CH)])
            return carry

        lax.fori_loop(0, RPT // CH, writeback, 0)
        plsc.subcore_barrier()

    count_phase(srcp, cid * N_PAD)
    count_phase(dstp, 2 * N_PAD + cid * N_PAD)


GRP = 8


@functools.partial(
    pl.kernel,
    out_type=jax.ShapeDtypeStruct((2 * N_PAD, M), jnp.float32),
    mesh=_MESH,
    scratch_types=[
        pltpu.VMEM_SHARED((N_PAD, M), jnp.float32),
        pltpu.VMEM((GRP, CH), jnp.int32),
        pltpu.VMEM((GRP, CH), jnp.int32),
        pltpu.VMEM((CH, M), jnp.float32),
        pltpu.VMEM((CH, M), jnp.float32),
        pltpu.SemaphoreType.DMA,
        pltpu.SemaphoreType.DMA,
        pltpu.SemaphoreType.DMA,
        pltpu.SemaphoreType.DMA,
    ],
)
def _spmm_kernel(zt, src2d, dst2d, p_out, acc, is_g, id_g, rb0, rb1,
                 sg0, sg1, ss0, ss1):
    cid = lax.axis_index("c")
    sid = lax.axis_index("s")
    wid = sid * 2 + cid
    RB = [rb0, rb1]
    SG = [sg0, sg1]
    SS = [ss0, ss1]
    zrow = jnp.zeros((16,), jnp.float32)

    def fill_zero(r, carry):
        for k in range(M // 16):
            rb0[r, pl.ds(k * 16, 16)] = zrow
        return carry

    lax.fori_loop(0, CH, fill_zero, 0)

    base_r = sid * RPT

    def zero_acc(b, carry):
        pltpu.sync_copy(rb0, acc.at[pl.ds(base_r + b * CH, CH)])
        return carry

    lax.fori_loop(0, RPT // CH, zero_acc, 0)
    plsc.subcore_barrier()

    def group_step(g, carry):
        row = wid * CPT + g * GRP
        pltpu.sync_copy(src2d.at[pl.ds(row, GRP)], is_g)
        pltpu.sync_copy(dst2d.at[pl.ds(row, GRP)], id_g)
        dg = [None] * GRP
        dsc = [None] * GRP
        for k in range(GRP):
            s = k % 2
            if k >= 2:
                dsc[k - 2].wait()
            dg[k] = pltpu.async_copy(zt.at[is_g.at[k]], RB[s], SG[s])
            if k >= 1:
                dg[k - 1].wait()
                dsc[k - 1] = pltpu.async_copy(
                    RB[(k - 1) % 2], acc.at[id_g.at[k - 1]],
                    SS[(k - 1) % 2], add=True)
        dg[GRP - 1].wait()
        dsc[GRP - 1] = pltpu.async_copy(
            RB[(GRP - 1) % 2], acc.at[id_g.at[GRP - 1]],
            SS[(GRP - 1) % 2], add=True)
        dsc[GRP - 2].wait()
        dsc[GRP - 1].wait()
        return carry

    lax.fori_loop(0, CPT // GRP, group_step, 0)
    plsc.subcore_barrier()

    def writeback(b, carry):
        off = base_r + b * CH
        pltpu.sync_copy(acc.at[pl.ds(off, CH)], rb0)
        pltpu.sync_copy(rb0, p_out.at[pl.ds(cid * N_PAD + off, CH)])
        return carry

    lax.fori_loop(0, RPT // CH, writeback, 0)


def _prep_body(xb, fw, do0, do1, di0, di1, xa_o, cb_o, gf_o):
    dego = do0[:, :1] + do1[:, :1]
    degi = di0[:, :1] + di1[:, :1]
    a = lax.rsqrt(jnp.maximum(dego, 1.0))
    b = lax.rsqrt(jnp.maximum(degi, 1.0))
    c = GAMMA * a * b
    xa_o[...] = xb[...].T * a
    cb_o[...] = jnp.broadcast_to(c, (M, M))
    g = lax.dot_general(fw[...], fw[...], (((0,), (0,)), ((), ())),
                        precision=lax.Precision.HIGHEST)
    nrm = jnp.sqrt(jnp.sum(g * g))
    gf_o[...] = g / (nrm + EPS_F)


def _prep_call(xp, f_w, degs):
    nb = N_PAD // M
    return pl.pallas_call(
        _prep_body,
        grid=(nb,),
        in_specs=[
            pl.BlockSpec((M, M), lambda i: (0, i)),
            pl.BlockSpec((M, M), lambda i: (0, 0)),
            pl.BlockSpec((M, M), lambda i: (i, 0)),
            pl.BlockSpec((M, M), lambda i, _nb=nb: (i + _nb, 0)),
            pl.BlockSpec((M, M), lambda i, _nb=nb: (i + 2 * _nb, 0)),
            pl.BlockSpec((M, M), lambda i, _nb=nb: (i + 3 * _nb, 0)),
        ],
        out_specs=[
            pl.BlockSpec((M, M), lambda i: (i, 0)),
            pl.BlockSpec((M, M), lambda i: (i, 0)),
            pl.BlockSpec((M, M), lambda i: (0, 0)),
        ],
        out_shape=[
            jax.ShapeDtypeStruct((N_PAD, M), jnp.float32),
            jax.ShapeDtypeStruct((N_PAD, M), jnp.float32),
            jax.ShapeDtypeStruct((M, M), jnp.float32),
        ],
        compiler_params=pltpu.CompilerParams(
            dimension_semantics=("arbitrary",)),
    )(xp, f_w, degs, degs, degs, degs)


def _update_body(p0, p1, cbk, xak, gf, z_o):
    acc = (p0[...] + p1[...]) * cbk[...]
    z_o[...] = lax.dot_general(
        acc, gf[...], (((1,), (0,)), ((), ())),
        precision=lax.Precision.HIGHEST) + xak[...]


def _update_call(pflat, cb, xa, gf):
    rb = 1024
    nb = N_PAD // rb
    return pl.pallas_call(
        _update_body,
        grid=(nb,),
        in_specs=[
            pl.BlockSpec((rb, M), lambda i: (i, 0)),
            pl.BlockSpec((rb, M), lambda i, _nb=nb: (i + _nb, 0)),
            pl.BlockSpec((rb, M), lambda i: (i, 0)),
            pl.BlockSpec((rb, M), lambda i: (i, 0)),
            pl.BlockSpec((M, M), lambda i: (0, 0)),
        ],
        out_specs=pl.BlockSpec((rb, M), lambda i: (i, 0)),
        out_shape=jax.ShapeDtypeStruct((N_PAD, M), jnp.float32),
        compiler_params=pltpu.CompilerParams(
            dimension_semantics=("arbitrary",)),
    )(pflat, pflat, cb, xa, gf)


def _final_body(zb, bw, o_ref):
    z = zb[...]
    nrm = jnp.maximum(jnp.sqrt(jnp.sum(z * z, axis=1, keepdims=True)), 1e-12)
    zn = z / nrm
    o_ref[...] = lax.dot_general(zn, bw[...], (((1,), (1,)), ((), ())),
                                 precision=lax.Precision.HIGHEST)


def _final_call(z, b_w):
    rb = 1000
    nb = N // rb
    return pl.pallas_call(
        _final_body,
        grid=(nb,),
        in_specs=[
            pl.BlockSpec((rb, M), lambda i: (i, 0)),
            pl.BlockSpec((MY, M), lambda i: (0, 0)),
        ],
        out_specs=pl.BlockSpec((rb, MY), lambda i: (i, 0)),
        out_shape=jax.ShapeDtypeStruct((N, MY), jnp.float32),
        compiler_params=pltpu.CompilerParams(
            dimension_semantics=("arbitrary",)),
    )(z, b_w)


def kernel(X, edge_index, F_w, B_w):
    src = edge_index[0]
    dst = edge_index[1]
    pad = jnp.full((E_PAD - E,), N, dtype=jnp.int32)
    srcp = jnp.concatenate([src, pad])
    dstp = jnp.concatenate([dst, pad])
    src2d = srcp.reshape(E_PAD // CH, CH)
    dst2d = dstp.reshape(E_PAD // CH, CH)
    xp = jnp.pad(X, ((0, 0), (0, N_PAD - N)))

    degs = _deg_kernel(srcp, dstp)
    xa, cb, gf = _prep_call(xp, F_w, degs)

    z = xa
    for _ in range(MAX_ITER):
        pflat = _spmm_kernel(z, src2d, dst2d)
        z = _update_call(pflat, cb, xa, gf)

    return _final_call(z, B_w)


# speedup vs baseline: 1.3473x; 1.0025x over previous
import functools

import jax
import jax.numpy as jnp
from jax import lax
from jax.experimental import pallas as pl
from jax.experimental.pallas import tpu as pltpu
from jax.experimental.pallas import tpu_sc as plsc

N = 10000
M = 128
MY = 16
E = 320000
MAX_ITER = 30
GAMMA = 0.8
EPS_F = 1e-6

N_PAD = 10240
CH = 128
N_TILES = 32
CPT = 80
E_PAD = N_TILES * CPT * CH
E_ALLOC = E_PAD + 2 * CH
RPT = N_PAD // 16

_MESH = plsc.VectorSubcoreMesh(core_axis_name="c", subcore_axis_name="s")


@functools.partial(
    pl.kernel,
    out_type=jax.ShapeDtypeStruct((4 * N_PAD, M), jnp.float32),
    mesh=_MESH,
    scratch_types=[
        pltpu.VMEM_SHARED((N_PAD, M), jnp.float32),
        pltpu.VMEM((CH,), jnp.int32),
        pltpu.VMEM((CH, M), jnp.float32),
        pltpu.VMEM((CH, M), jnp.float32),
    ],
)
def _deg_kernel(srcp, dstp, deg_out, acc, idx_v, ones_v, bnc_v):
    cid = lax.axis_index("c")
    sid = lax.axis_index("s")
    wid = sid * 2 + cid
    onerow = jnp.ones((16,), jnp.float32)
    zrow = jnp.zeros((16,), jnp.float32)

    def fill_ones(r, carry):
        for k in range(M // 16):
            ones_v[r, pl.ds(k * 16, 16)] = onerow
        return carry

    lax.fori_loop(0, CH, fill_ones, 0)

    def fill_bnc_zero(r, carry):
        for k in range(M // 16):
            bnc_v[r, pl.ds(k * 16, 16)] = zrow
        return carry

    base_r = sid * RPT

    def zero_acc(b, carry):
        pltpu.sync_copy(bnc_v, acc.at[pl.ds(base_r + b * CH, CH)])
        return carry

    def count_phase(idx_hbm, out_base):
        lax.fori_loop(0, CH, fill_bnc_zero, 0)
        lax.fori_loop(0, RPT // CH, zero_acc, 0)
        plsc.subcore_barrier()

        def edge_step(j, carry):
            base = (wid * CPT + j) * CH
            pltpu.sync_copy(idx_hbm.at[pl.ds(base, CH)], idx_v)
            pltpu.sync_copy(ones_v, acc.at[idx_v], add=True)
            return carry

        lax.fori_loop(0, CPT, edge_step, 0)
        plsc.subcore_barrier()

        def writeback(b, carry):
            off = base_r + b * CH
            pltpu.sync_copy(acc.at[pl.ds(off, CH)], bnc_v)
            pltpu.sync_copy(bnc_v, deg_out.at[pl.ds(out_base + off, CH)])
            return carry

        lax.fori_loop(0, RPT // CH, writeback, 0)
        plsc.subcore_barrier()

    count_phase(srcp, cid * N_PAD)
    count_phase(dstp, 2 * N_PAD + cid * N_PAD)


GRP = 8


@functools.partial(
    pl.kernel,
    out_type=jax.ShapeDtypeStruct((2 * N_PAD, M), jnp.float32),
    mesh=_MESH,
    scratch_types=[
        pltpu.VMEM_SHARED((N_PAD, M), jnp.float32),
        pltpu.VMEM((GRP, CH), jnp.int32),
        pltpu.VMEM((GRP, CH), jnp.int32),
        pltpu.VMEM((CH, M), jnp.float32),
        pltpu.VMEM((CH, M), jnp.float32),
        pltpu.SemaphoreType.DMA,
        pltpu.SemaphoreType.DMA,
        pltpu.SemaphoreType.DMA,
        pltpu.SemaphoreType.DMA,
    ],
)
def _spmm_kernel(zt, src2d, dst2d, p_out, acc, is_g, id_g, rb0, rb1,
                 sg0, sg1, ss0, ss1):
    cid = lax.axis_index("c")
    sid = lax.axis_index("s")
    wid = sid * 2 + cid
    RB = [rb0, rb1]
    SG = [sg0, sg1]
    SS = [ss0, ss1]
    zrow = jnp.zeros((16,), jnp.float32)

    def fill_zero(r, carry):
        for k in range(M // 16):
            rb0[r, pl.ds(k * 16, 16)] = zrow
        return carry

    lax.fori_loop(0, CH, fill_zero, 0)

    base_r = sid * RPT
    dz = [pltpu.async_copy(rb0, acc.at[pl.ds(base_r + b * CH, CH)], sg0)
          for b in range(RPT // CH)]
    for d in dz:
        d.wait()
    plsc.subcore_barrier()

    def group_step(g, carry):
        row = wid * CPT + g * GRP
        pltpu.sync_copy(src2d.at[pl.ds(row, GRP)], is_g)
        pltpu.sync_copy(dst2d.at[pl.ds(row, GRP)], id_g)
        dg = [None] * GRP
        dsc = [None] * GRP
        for k in range(GRP):
            s = k % 2
            if k >= 2:
                dsc[k - 2].wait()
            dg[k] = pltpu.async_copy(zt.at[is_g.at[k]], RB[s], SG[s])
            if k >= 1:
                dg[k - 1].wait()
                dsc[k - 1] = pltpu.async_copy(
                    RB[(k - 1) % 2], acc.at[id_g.at[k - 1]],
                    SS[(k - 1) % 2], add=True)
        dg[GRP - 1].wait()
        dsc[GRP - 1] = pltpu.async_copy(
            RB[(GRP - 1) % 2], acc.at[id_g.at[GRP - 1]],
            SS[(GRP - 1) % 2], add=True)
        dsc[GRP - 2].wait()
        dsc[GRP - 1].wait()
        return carry

    lax.fori_loop(0, CPT // GRP, group_step, 0)
    plsc.subcore_barrier()

    din = [None] * (RPT // CH)
    dout = [None] * (RPT // CH)
    for b in range(RPT // CH):
        s = b % 2
        off = base_r + b * CH
        if b >= 2:
            dout[b - 2].wait()
        din[b] = pltpu.async_copy(acc.at[pl.ds(off, CH)], RB[s], SG[s])
        din[b].wait()
        dout[b] = pltpu.async_copy(
            RB[s], p_out.at[pl.ds(cid * N_PAD + off, CH)], SS[s])
    dout[RPT // CH - 2].wait()
    dout[RPT // CH - 1].wait()


def _prep_body(xb, fw, do0, do1, di0, di1, xa_o, cb_o, gf_o):
    dego = do0[:, :1] + do1[:, :1]
    degi = di0[:, :1] + di1[:, :1]
    a = lax.rsqrt(jnp.maximum(dego, 1.0))
    b = lax.rsqrt(jnp.maximum(degi, 1.0))
    c = GAMMA * a * b
    xa_o[...] = xb[...].T * a
    cb_o[...] = jnp.broadcast_to(c, (M, M))
    g = lax.dot_general(fw[...], fw[...], (((0,), (0,)), ((), ())),
                        precision=lax.Precision.HIGHEST)
    nrm = jnp.sqrt(jnp.sum(g * g))
    gf_o[...] = g / (nrm + EPS_F)


def _prep_call(xp, f_w, degs):
    nb = N_PAD // M
    return pl.pallas_call(
        _prep_body,
        grid=(nb,),
        in_specs=[
            pl.BlockSpec((M, M), lambda i: (0, i)),
            pl.BlockSpec((M, M), lambda i: (0, 0)),
            pl.BlockSpec((M, M), lambda i: (i, 0)),
            pl.BlockSpec((M, M), lambda i, _nb=nb: (i + _nb, 0)),
            pl.BlockSpec((M, M), lambda i, _nb=nb: (i + 2 * _nb, 0)),
            pl.BlockSpec((M, M), lambda i, _nb=nb: (i + 3 * _nb, 0)),
        ],
        out_specs=[
            pl.BlockSpec((M, M), lambda i: (i, 0)),
            pl.BlockSpec((M, M), lambda i: (i, 0)),
            pl.BlockSpec((M, M), lambda i: (0, 0)),
        ],
        out_shape=[
            jax.ShapeDtypeStruct((N_PAD, M), jnp.float32),
            jax.ShapeDtypeStruct((N_PAD, M), jnp.float32),
            jax.ShapeDtypeStruct((M, M), jnp.float32),
        ],
        compiler_params=pltpu.CompilerParams(
            dimension_semantics=("arbitrary",)),
    )(xp, f_w, degs, degs, degs, degs)


def _update_body(p0, p1, cbk, xak, gf, z_o):
    acc = (p0[...] + p1[...]) * cbk[...]
    z_o[...] = lax.dot_general(
        acc, gf[...], (((1,), (0,)), ((), ())),
        precision=lax.Precision.HIGHEST) + xak[...]


def _update_call(pflat, cb, xa, gf):
    rb = 1024
    nb = N_PAD // rb
    return pl.pallas_call(
        _update_body,
        grid=(nb,),
        in_specs=[
            pl.BlockSpec((rb, M), lambda i: (i, 0)),
            pl.BlockSpec((rb, M), lambda i, _nb=nb: (i + _nb, 0)),
            pl.BlockSpec((rb, M), lambda i: (i, 0)),
            pl.BlockSpec((rb, M), lambda i: (i, 0)),
            pl.BlockSpec((M, M), lambda i: (0, 0)),
        ],
        out_specs=pl.BlockSpec((rb, M), lambda i: (i, 0)),
        out_shape=jax.ShapeDtypeStruct((N_PAD, M), jnp.float32),
        compiler_params=pltpu.CompilerParams(
            dimension_semantics=("arbitrary",)),
    )(pflat, pflat, cb, xa, gf)


def _final_body(zb, bw, o_ref):
    z = zb[...]
    nrm = jnp.maximum(jnp.sqrt(jnp.sum(z * z, axis=1, keepdims=True)), 1e-12)
    zn = z / nrm
    o_ref[...] = lax.dot_general(zn, bw[...], (((1,), (1,)), ((), ())),
                                 precision=lax.Precision.HIGHEST)


def _final_call(z, b_w):
    rb = 1000
    nb = N // rb
    return pl.pallas_call(
        _final_body,
        grid=(nb,),
        in_specs=[
            pl.BlockSpec((rb, M), lambda i: (i, 0)),
            pl.BlockSpec((MY, M), lambda i: (0, 0)),
        ],
        out_specs=pl.BlockSpec((rb, MY), lambda i: (i, 0)),
        out_shape=jax.ShapeDtypeStruct((N, MY), jnp.float32),
        compiler_params=pltpu.CompilerParams(
            dimension_semantics=("arbitrary",)),
    )(z, b_w)


def kernel(X, edge_index, F_w, B_w):
    src = edge_index[0]
    dst = edge_index[1]
    pad = jnp.full((E_PAD - E,), N, dtype=jnp.int32)
    srcp = jnp.concatenate([src, pad])
    dstp = jnp.concatenate([dst, pad])
    src2d = srcp.reshape(E_PAD // CH, CH)
    dst2d = dstp.reshape(E_PAD // CH, CH)
    xp = jnp.pad(X, ((0, 0), (0, N_PAD - N)))

    degs = _deg_kernel(srcp, dstp)
    xa, cb, gf = _prep_call(xp, F_w, degs)

    z = xa
    for _ in range(MAX_ITER):
        pflat = _spmm_kernel(z, src2d, dst2d)
        z = _update_call(pflat, cb, xa, gf)

    return _final_call(z, B_w)
